# baseline (speedup 1.0000x reference)
import functools

import jax
import jax.numpy as jnp
from jax import lax
from jax.experimental import pallas as pl
from jax.experimental.pallas import tpu as pltpu
from jax.experimental.pallas import tpu_sc as plsc

N_NODES = 50000
D_FEAT = 256
G_DIM = 256
NUM_GRAPHS = 256
UNITS = 256
GROUPS = 8
EPS = 1e-3

BLK = 5000
NBLK = N_NODES // BLK
NEG = -1e30


def _pass1_body(x_ref, idx_ref, glob_ref, w0_ref, b0_ref, g0_ref, be0_ref,
                w1_ref, b1_ref, g1_ref, be1_ref, w2_ref, b2_ref,
                logits_ref, s_ref,
                cg_sc, w0a_sc, w1_sc, p_sc, b0_sc, b1_sc, s_sc):
    k = pl.program_id(0)

    @pl.when(k == 0)
    def _init():
        gs = UNITS // GROUPS
        ri = jax.lax.broadcasted_iota(jnp.int32, (UNITS, UNITS), 0) // gs
        ci = jax.lax.broadcasted_iota(jnp.int32, (UNITS, UNITS), 1) // gs
        p = (ri == ci).astype(jnp.float32) * (1.0 / gs)
        p_sc[...] = p.astype(jnp.bfloat16)
        cg = jnp.dot(glob_ref[...], w0_ref[D_FEAT:, :],
                     preferred_element_type=jnp.float32)
        cg_sc[...] = (cg - jnp.dot(cg, p, preferred_element_type=jnp.float32)
                      ).astype(jnp.bfloat16)
        w0a = w0_ref[:D_FEAT, :]
        w0a_sc[...] = w0a - jnp.dot(w0a, p, preferred_element_type=jnp.float32)
        w1 = w1_ref[...]
        w1_sc[...] = w1 - jnp.dot(w1, p, preferred_element_type=jnp.float32)
        b0_sc[...] = b0_ref[...] - jnp.dot(b0_ref[...], p,
                                           preferred_element_type=jnp.float32)
        b1_sc[...] = b1_ref[...] - jnp.dot(b1_ref[...], p,
                                           preferred_element_type=jnp.float32)
        s_sc[...] = jnp.zeros((1, NUM_GRAPHS), jnp.float32)

    x = x_ref[...]
    idx = idx_ref[0]
    oh_b = idx == jax.lax.broadcasted_iota(jnp.int32, (BLK, NUM_GRAPHS), 1)
    oh = oh_b.astype(jnp.bfloat16)

    hc = (jnp.dot(x, w0a_sc[...], preferred_element_type=jnp.float32)
          + jnp.dot(oh, cg_sc[...], preferred_element_type=jnp.float32)
          + b0_sc[...])
    var = jnp.dot((hc * hc).astype(jnp.bfloat16), p_sc[...],
                  preferred_element_type=jnp.float32)
    h = jnp.maximum(hc * jax.lax.rsqrt(var + EPS) * g0_ref[...] + be0_ref[...],
                    0.0)
    hc = jnp.dot(h, w1_sc[...], preferred_element_type=jnp.float32) + b1_sc[...]
    var = jnp.dot((hc * hc).astype(jnp.bfloat16), p_sc[...],
                  preferred_element_type=jnp.float32)
    h = jnp.maximum(hc * jax.lax.rsqrt(var + EPS) * g1_ref[...] + be1_ref[...],
                    0.0)
    l = jnp.dot(h, w2_ref[...], preferred_element_type=jnp.float32) + b2_ref[0, 0]
    logits_ref[...] = l

    e_node = jnp.exp(l)
    s_add = jax.lax.dot_general(e_node.astype(jnp.bfloat16), oh,
                                (((0,), (0,)), ((), ())),
                                preferred_element_type=jnp.float32)
    s_new = s_sc[...] + s_add
    s_sc[...] = s_new
    s_ref[...] = s_new


_SC_NW = 32
_SC_LANES = 16
_SC_CHUNK = 1568
_SC_TAIL = N_NODES - (_SC_NW - 1) * _SC_CHUNK


def _sc_norm_body(l_hbm, idx_hbm, s_hbm, att_hbm,
                  l_v, idx_v, att_v, s_v):
    wid = lax.axis_index("s") * 2 + lax.axis_index("c")
    base = wid * _SC_CHUNK
    pltpu.sync_copy(s_hbm, s_v)

    def step(i, _):
        sl = pl.ds(i * _SC_LANES, _SC_LANES)
        iv = idx_v[sl]
        sg = plsc.load_gather(s_v, [iv])
        att_v[sl] = jnp.exp(l_v[sl]) / sg
        return 0

    def run(n):
        pltpu.sync_copy(l_hbm.at[pl.ds(base, n)], l_v.at[pl.ds(0, n)])
        pltpu.sync_copy(idx_hbm.at[pl.ds(base, n)], idx_v.at[pl.ds(0, n)])
        lax.fori_loop(0, n // _SC_LANES, step, 0)
        pltpu.sync_copy(att_v.at[pl.ds(0, n)], att_hbm.at[pl.ds(base, n)])

    @pl.when(wid < _SC_NW - 1)
    def _full():
        run(_SC_CHUNK)

    @pl.when(wid == _SC_NW - 1)
    def _tail():
        run(_SC_TAIL)


_sc_norm = pl.kernel(
    _sc_norm_body,
    out_type=jax.ShapeDtypeStruct((N_NODES,), jnp.float32),
    mesh=plsc.VectorSubcoreMesh(core_axis_name="c", subcore_axis_name="s"),
    scratch_types=[
        pltpu.VMEM((_SC_CHUNK,), jnp.float32),
        pltpu.VMEM((_SC_CHUNK,), jnp.int32),
        pltpu.VMEM((_SC_CHUNK,), jnp.float32),
        pltpu.VMEM((NUM_GRAPHS,), jnp.float32),
    ],
    compiler_params=pltpu.CompilerParams(needs_layout_passes=False),
)


@jax.jit
def kernel(inputs, graph_indices, global_features, W0, b0, gamma0, beta0,
           W1, b1, gamma1, beta1, W2, b2):
    idx3 = graph_indices.astype(jnp.int32).reshape(NBLK, BLK, 1)
    row = lambda v: v.reshape(1, -1)

    logits, s = pl.pallas_call(
        _pass1_body,
        grid=(NBLK,),
        in_specs=[
            pl.BlockSpec((BLK, D_FEAT), lambda k: (k, 0)),
            pl.BlockSpec((1, BLK, 1), lambda k: (k, 0, 0)),
            pl.BlockSpec((NUM_GRAPHS, G_DIM), lambda k: (0, 0)),
            pl.BlockSpec((D_FEAT + G_DIM, UNITS), lambda k: (0, 0)),
            pl.BlockSpec((1, UNITS), lambda k: (0, 0)),
            pl.BlockSpec((1, UNITS), lambda k: (0, 0)),
            pl.BlockSpec((1, UNITS), lambda k: (0, 0)),
            pl.BlockSpec((UNITS, UNITS), lambda k: (0, 0)),
            pl.BlockSpec((1, UNITS), lambda k: (0, 0)),
            pl.BlockSpec((1, UNITS), lambda k: (0, 0)),
            pl.BlockSpec((1, UNITS), lambda k: (0, 0)),
            pl.BlockSpec((UNITS, 1), lambda k: (0, 0)),
            pl.BlockSpec((1, 1), lambda k: (0, 0)),
        ],
        out_specs=[
            pl.BlockSpec((BLK, 1), lambda k: (k, 0)),
            pl.BlockSpec((1, NUM_GRAPHS), lambda k: (0, 0)),
        ],
        out_shape=[
            jax.ShapeDtypeStruct((N_NODES, 1), jnp.float32),
            jax.ShapeDtypeStruct((1, NUM_GRAPHS), jnp.float32),
        ],
        scratch_shapes=[
            pltpu.VMEM((G_DIM, UNITS), jnp.bfloat16),
            pltpu.VMEM((D_FEAT, UNITS), jnp.float32),
            pltpu.VMEM((UNITS, UNITS), jnp.float32),
            pltpu.VMEM((UNITS, UNITS), jnp.bfloat16),
            pltpu.VMEM((1, UNITS), jnp.float32),
            pltpu.VMEM((1, UNITS), jnp.float32),
            pltpu.VMEM((1, NUM_GRAPHS), jnp.float32),
        ],
        compiler_params=pltpu.CompilerParams(
            dimension_semantics=("arbitrary",)),
    )(inputs, idx3, global_features, W0, row(b0), row(gamma0), row(beta0),
      W1, row(b1), row(gamma1), row(beta1), W2, b2.reshape(1, 1))

    att = _sc_norm(logits.reshape(N_NODES), graph_indices.astype(jnp.int32),
                   s.reshape(NUM_GRAPHS))
    return att[:, None]


# speedup vs baseline: 1.0245x; 1.0245x over previous
import functools

import jax
import jax.numpy as jnp
from jax import lax
from jax.experimental import pallas as pl
from jax.experimental.pallas import tpu as pltpu
from jax.experimental.pallas import tpu_sc as plsc

N_NODES = 50000
D_FEAT = 256
G_DIM = 256
NUM_GRAPHS = 256
UNITS = 256
GROUPS = 8
EPS = 1e-3

BLK = 5000
NBLK = N_NODES // BLK
NEG = -1e30


def _pass1_body(x_ref, idx_ref, glob_ref, w0_ref, b0_ref, g0_ref, be0_ref,
                w1_ref, b1_ref, g1_ref, be1_ref, w2_ref, b2_ref,
                logits_ref, s_ref,
                cg_sc, w0a_sc, w1_sc, p_sc, b0_sc, b1_sc, s_sc):
    k = pl.program_id(0)

    @pl.when(k == 0)
    def _init():
        gs = UNITS // GROUPS
        ri = jax.lax.broadcasted_iota(jnp.int32, (UNITS, UNITS), 0) // gs
        ci = jax.lax.broadcasted_iota(jnp.int32, (UNITS, UNITS), 1) // gs
        p = (ri == ci).astype(jnp.float32) * (1.0 / gs)
        p_sc[...] = p
        cg = jnp.dot(glob_ref[...], w0_ref[D_FEAT:, :],
                     preferred_element_type=jnp.float32)
        cg_sc[...] = (cg - jnp.dot(cg, p, preferred_element_type=jnp.float32)
                      ).astype(jnp.bfloat16)
        w0a = w0_ref[:D_FEAT, :]
        w0a_sc[...] = w0a - jnp.dot(w0a, p, preferred_element_type=jnp.float32)
        w1 = w1_ref[...]
        w1_sc[...] = w1 - jnp.dot(w1, p, preferred_element_type=jnp.float32)
        b0_sc[...] = b0_ref[...] - jnp.dot(b0_ref[...], p,
                                           preferred_element_type=jnp.float32)
        b1_sc[...] = b1_ref[...] - jnp.dot(b1_ref[...], p,
                                           preferred_element_type=jnp.float32)
        s_sc[...] = jnp.zeros((1, NUM_GRAPHS), jnp.float32)

    x = x_ref[...]
    idx = idx_ref[0]
    oh_b = idx == jax.lax.broadcasted_iota(jnp.int32, (BLK, NUM_GRAPHS), 1)
    oh = oh_b.astype(jnp.bfloat16)

    hc = (jnp.dot(x, w0a_sc[...], preferred_element_type=jnp.float32)
          + jnp.dot(oh, cg_sc[...], preferred_element_type=jnp.float32)
          + b0_sc[...])
    var = jnp.dot(hc * hc, p_sc[...], preferred_element_type=jnp.float32)
    h = jnp.maximum(hc * jax.lax.rsqrt(var + EPS) * g0_ref[...] + be0_ref[...],
                    0.0)
    hc = jnp.dot(h, w1_sc[...], preferred_element_type=jnp.float32) + b1_sc[...]
    var = jnp.dot(hc * hc, p_sc[...], preferred_element_type=jnp.float32)
    h = jnp.maximum(hc * jax.lax.rsqrt(var + EPS) * g1_ref[...] + be1_ref[...],
                    0.0)
    l = jnp.dot(h, w2_ref[...], preferred_element_type=jnp.float32) + b2_ref[0, 0]
    logits_ref[...] = l

    e_node = jnp.exp(l)
    s_add = jax.lax.dot_general(e_node.astype(jnp.bfloat16), oh,
                                (((0,), (0,)), ((), ())),
                                preferred_element_type=jnp.float32)
    s_new = s_sc[...] + s_add
    s_sc[...] = s_new
    s_ref[...] = s_new


_SC_NW = 32
_SC_LANES = 16
_SC_CHUNK = 1568
_SC_TAIL = N_NODES - (_SC_NW - 1) * _SC_CHUNK


def _sc_norm_body(l_hbm, idx_hbm, s_hbm, att_hbm,
                  l_v, idx_v, att_v, s_v):
    wid = lax.axis_index("s") * 2 + lax.axis_index("c")
    base = wid * _SC_CHUNK
    pltpu.sync_copy(s_hbm, s_v)

    _UNROLL = 7

    def step(j, _):
        def one(i):
            sl = pl.ds(i * _SC_LANES, _SC_LANES)
            iv = idx_v[sl]
            sg = plsc.load_gather(s_v, [iv])
            att_v[sl] = jnp.exp(l_v[sl]) / sg
        for u in range(_UNROLL):
            one(j * _UNROLL + u)
        return 0

    def run(n):
        pltpu.sync_copy(l_hbm.at[pl.ds(base, n)], l_v.at[pl.ds(0, n)])
        pltpu.sync_copy(idx_hbm.at[pl.ds(base, n)], idx_v.at[pl.ds(0, n)])
        lax.fori_loop(0, n // (_SC_LANES * _UNROLL), step, 0)
        for r in range(n // (_SC_LANES * _UNROLL) * _UNROLL, n // _SC_LANES):
            def one(i):
                sl = pl.ds(i * _SC_LANES, _SC_LANES)
                iv = idx_v[sl]
                sg = plsc.load_gather(s_v, [iv])
                att_v[sl] = jnp.exp(l_v[sl]) / sg
            one(r)
        pltpu.sync_copy(att_v.at[pl.ds(0, n)], att_hbm.at[pl.ds(base, n)])

    @pl.when(wid < _SC_NW - 1)
    def _full():
        run(_SC_CHUNK)

    @pl.when(wid == _SC_NW - 1)
    def _tail():
        run(_SC_TAIL)


_sc_norm = pl.kernel(
    _sc_norm_body,
    out_type=jax.ShapeDtypeStruct((N_NODES,), jnp.float32),
    mesh=plsc.VectorSubcoreMesh(core_axis_name="c", subcore_axis_name="s"),
    scratch_types=[
        pltpu.VMEM((_SC_CHUNK,), jnp.float32),
        pltpu.VMEM((_SC_CHUNK,), jnp.int32),
        pltpu.VMEM((_SC_CHUNK,), jnp.float32),
        pltpu.VMEM((NUM_GRAPHS,), jnp.float32),
    ],
    compiler_params=pltpu.CompilerParams(needs_layout_passes=False),
)


@jax.jit
def kernel(inputs, graph_indices, global_features, W0, b0, gamma0, beta0,
           W1, b1, gamma1, beta1, W2, b2):
    idx3 = graph_indices.astype(jnp.int32).reshape(NBLK, BLK, 1)
    row = lambda v: v.reshape(1, -1)

    logits, s = pl.pallas_call(
        _pass1_body,
        grid=(NBLK,),
        in_specs=[
            pl.BlockSpec((BLK, D_FEAT), lambda k: (k, 0)),
            pl.BlockSpec((1, BLK, 1), lambda k: (k, 0, 0)),
            pl.BlockSpec((NUM_GRAPHS, G_DIM), lambda k: (0, 0)),
            pl.BlockSpec((D_FEAT + G_DIM, UNITS), lambda k: (0, 0)),
            pl.BlockSpec((1, UNITS), lambda k: (0, 0)),
            pl.BlockSpec((1, UNITS), lambda k: (0, 0)),
            pl.BlockSpec((1, UNITS), lambda k: (0, 0)),
            pl.BlockSpec((UNITS, UNITS), lambda k: (0, 0)),
            pl.BlockSpec((1, UNITS), lambda k: (0, 0)),
            pl.BlockSpec((1, UNITS), lambda k: (0, 0)),
            pl.BlockSpec((1, UNITS), lambda k: (0, 0)),
            pl.BlockSpec((UNITS, 1), lambda k: (0, 0)),
            pl.BlockSpec((1, 1), lambda k: (0, 0)),
        ],
        out_specs=[
            pl.BlockSpec((BLK, 1), lambda k: (k, 0)),
            pl.BlockSpec((1, NUM_GRAPHS), lambda k: (0, 0)),
        ],
        out_shape=[
            jax.ShapeDtypeStruct((N_NODES, 1), jnp.float32),
            jax.ShapeDtypeStruct((1, NUM_GRAPHS), jnp.float32),
        ],
        scratch_shapes=[
            pltpu.VMEM((G_DIM, UNITS), jnp.bfloat16),
            pltpu.VMEM((D_FEAT, UNITS), jnp.float32),
            pltpu.VMEM((UNITS, UNITS), jnp.float32),
            pltpu.VMEM((UNITS, UNITS), jnp.float32),
            pltpu.VMEM((1, UNITS), jnp.float32),
            pltpu.VMEM((1, UNITS), jnp.float32),
            pltpu.VMEM((1, NUM_GRAPHS), jnp.float32),
        ],
        compiler_params=pltpu.CompilerParams(
            dimension_semantics=("arbitrary",)),
    )(inputs, idx3, global_features, W0, row(b0), row(gamma0), row(beta0),
      W1, row(b1), row(gamma1), row(beta1), W2, b2.reshape(1, 1))

    att = _sc_norm(logits.reshape(N_NODES), graph_indices.astype(jnp.int32),
                   s.reshape(NUM_GRAPHS))
    return att[:, None]


# speedup vs baseline: 1.0433x; 1.0183x over previous
import functools

import jax
import jax.numpy as jnp
from jax import lax
from jax.experimental import pallas as pl
from jax.experimental.pallas import tpu as pltpu
from jax.experimental.pallas import tpu_sc as plsc

N_NODES = 50000
D_FEAT = 256
G_DIM = 256
NUM_GRAPHS = 256
UNITS = 256
GROUPS = 8
EPS = 1e-3

BLK = 5000
NBLK = N_NODES // BLK
NEG = -1e30


def _pass1_body(x_ref, idx_ref, glob_ref, w0_ref, b0_ref, g0_ref, be0_ref,
                w1_ref, b1_ref, g1_ref, be1_ref, w2_ref, b2_ref,
                logits_ref,
                cg_sc, w0a_sc, w1_sc, p_sc, b0_sc, b1_sc):
    k = pl.program_id(0)

    @pl.when(k == 0)
    def _init():
        gs = UNITS // GROUPS
        ri = jax.lax.broadcasted_iota(jnp.int32, (UNITS, UNITS), 0) // gs
        ci = jax.lax.broadcasted_iota(jnp.int32, (UNITS, UNITS), 1) // gs
        p = (ri == ci).astype(jnp.float32) * (1.0 / gs)
        p_sc[...] = p
        cg = jnp.dot(glob_ref[...], w0_ref[D_FEAT:, :],
                     preferred_element_type=jnp.float32)
        cg_sc[...] = (cg - jnp.dot(cg, p, preferred_element_type=jnp.float32)
                      ).astype(jnp.bfloat16)
        w0a = w0_ref[:D_FEAT, :]
        w0a_sc[...] = w0a - jnp.dot(w0a, p, preferred_element_type=jnp.float32)
        w1 = w1_ref[...]
        w1_sc[...] = w1 - jnp.dot(w1, p, preferred_element_type=jnp.float32)
        b0_sc[...] = b0_ref[...] - jnp.dot(b0_ref[...], p,
                                           preferred_element_type=jnp.float32)
        b1_sc[...] = b1_ref[...] - jnp.dot(b1_ref[...], p,
                                           preferred_element_type=jnp.float32)

    x = x_ref[...]
    idx = idx_ref[0]
    oh_b = idx == jax.lax.broadcasted_iota(jnp.int32, (BLK, NUM_GRAPHS), 1)
    oh = oh_b.astype(jnp.bfloat16)

    hc = (jnp.dot(x, w0a_sc[...], preferred_element_type=jnp.float32)
          + jnp.dot(oh, cg_sc[...], preferred_element_type=jnp.float32)
          + b0_sc[...])
    var = jnp.dot(hc * hc, p_sc[...], preferred_element_type=jnp.float32)
    h = jnp.maximum(hc * jax.lax.rsqrt(var + EPS) * g0_ref[...] + be0_ref[...],
                    0.0)
    hc = jnp.dot(h, w1_sc[...], preferred_element_type=jnp.float32) + b1_sc[...]
    var = jnp.dot(hc * hc, p_sc[...], preferred_element_type=jnp.float32)
    h = jnp.maximum(hc * jax.lax.rsqrt(var + EPS) * g1_ref[...] + be1_ref[...],
                    0.0)
    l = jnp.dot(h, w2_ref[...], preferred_element_type=jnp.float32) + b2_ref[0, 0]
    logits_ref[...] = l


_SC_NW = 32
_SC_NS = 16
_SC_LANES = 16
_SC_CHUNK = 1568
_SC_TAIL = N_NODES - (_SC_NW - 1) * _SC_CHUNK
_SC_SCHUNK = 3136
_SC_STAIL = N_NODES - (_SC_NS - 1) * _SC_SCHUNK
_SC_UNROLL = 7


def _sc_softmax_body(l_hbm, idx_hbm, att_hbm,
                     l_v, idx_v, att_v, s_v, l2_v, idx2_v, stab_v, red_v,
                     shared_v):
    cid = lax.axis_index("c")
    sid = lax.axis_index("s")
    wid = sid * 2 + cid

    for i in range(NUM_GRAPHS // _SC_LANES):
        stab_v[pl.ds(i * _SC_LANES, _SC_LANES)] = jnp.zeros((_SC_LANES,),
                                                            jnp.float32)

    def sum_one(i):
        sl = pl.ds(i * _SC_LANES, _SC_LANES)
        e = jnp.exp(l2_v[sl])
        plsc.addupdate_scatter(stab_v, [idx2_v[sl]], e)

    def sum_step(j, _):
        for u in range(_SC_UNROLL):
            sum_one(j * _SC_UNROLL + u)
        return 0

    def sum_run(n):
        sbase = sid * _SC_SCHUNK
        pltpu.sync_copy(l_hbm.at[pl.ds(sbase, n)], l2_v.at[pl.ds(0, n)])
        pltpu.sync_copy(idx_hbm.at[pl.ds(sbase, n)], idx2_v.at[pl.ds(0, n)])
        nv = n // _SC_LANES
        lax.fori_loop(0, nv // _SC_UNROLL, sum_step, 0)
        for r in range(nv // _SC_UNROLL * _SC_UNROLL, nv):
            sum_one(r)

    @pl.when(sid < _SC_NS - 1)
    def _sfull():
        sum_run(_SC_SCHUNK)

    @pl.when(sid == _SC_NS - 1)
    def _stail():
        sum_run(_SC_STAIL)

    pltpu.sync_copy(stab_v, shared_v.at[sid])
    plsc.subcore_barrier()
    pltpu.sync_copy(shared_v, red_v)
    for i in range(NUM_GRAPHS // _SC_LANES):
        sl = pl.ds(i * _SC_LANES, _SC_LANES)
        acc = red_v[0, sl]
        for r in range(1, _SC_NS):
            acc = acc + red_v[r, sl]
        s_v[sl] = acc

    def norm_one(i):
        sl = pl.ds(i * _SC_LANES, _SC_LANES)
        sg = plsc.load_gather(s_v, [idx_v[sl]])
        att_v[sl] = jnp.exp(l_v[sl]) / sg

    def norm_step(j, _):
        for u in range(_SC_UNROLL):
            norm_one(j * _SC_UNROLL + u)
        return 0

    def norm_run(n):
        base = wid * _SC_CHUNK
        pltpu.sync_copy(l_hbm.at[pl.ds(base, n)], l_v.at[pl.ds(0, n)])
        pltpu.sync_copy(idx_hbm.at[pl.ds(base, n)], idx_v.at[pl.ds(0, n)])
        nv = n // _SC_LANES
        lax.fori_loop(0, nv // _SC_UNROLL, norm_step, 0)
        for r in range(nv // _SC_UNROLL * _SC_UNROLL, nv):
            norm_one(r)
        pltpu.sync_copy(att_v.at[pl.ds(0, n)], att_hbm.at[pl.ds(base, n)])

    @pl.when(wid < _SC_NW - 1)
    def _nfull():
        norm_run(_SC_CHUNK)

    @pl.when(wid == _SC_NW - 1)
    def _ntail():
        norm_run(_SC_TAIL)


_sc_softmax = pl.kernel(
    _sc_softmax_body,
    out_type=jax.ShapeDtypeStruct((N_NODES,), jnp.float32),
    mesh=plsc.VectorSubcoreMesh(core_axis_name="c", subcore_axis_name="s"),
    scratch_types=[
        pltpu.VMEM((_SC_CHUNK,), jnp.float32),
        pltpu.VMEM((_SC_CHUNK,), jnp.int32),
        pltpu.VMEM((_SC_CHUNK,), jnp.float32),
        pltpu.VMEM((NUM_GRAPHS,), jnp.float32),
        pltpu.VMEM((_SC_SCHUNK,), jnp.float32),
        pltpu.VMEM((_SC_SCHUNK,), jnp.int32),
        pltpu.VMEM((NUM_GRAPHS,), jnp.float32),
        pltpu.VMEM((_SC_NS, NUM_GRAPHS), jnp.float32),
        pltpu.VMEM_SHARED((_SC_NS, NUM_GRAPHS), jnp.float32),
    ],
    compiler_params=pltpu.CompilerParams(needs_layout_passes=False),
)


@jax.jit
def kernel(inputs, graph_indices, global_features, W0, b0, gamma0, beta0,
           W1, b1, gamma1, beta1, W2, b2):
    idx3 = graph_indices.astype(jnp.int32).reshape(NBLK, BLK, 1)
    row = lambda v: v.reshape(1, -1)

    logits = pl.pallas_call(
        _pass1_body,
        grid=(NBLK,),
        in_specs=[
            pl.BlockSpec((BLK, D_FEAT), lambda k: (k, 0)),
            pl.BlockSpec((1, BLK, 1), lambda k: (k, 0, 0)),
            pl.BlockSpec((NUM_GRAPHS, G_DIM), lambda k: (0, 0)),
            pl.BlockSpec((D_FEAT + G_DIM, UNITS), lambda k: (0, 0)),
            pl.BlockSpec((1, UNITS), lambda k: (0, 0)),
            pl.BlockSpec((1, UNITS), lambda k: (0, 0)),
            pl.BlockSpec((1, UNITS), lambda k: (0, 0)),
            pl.BlockSpec((UNITS, UNITS), lambda k: (0, 0)),
            pl.BlockSpec((1, UNITS), lambda k: (0, 0)),
            pl.BlockSpec((1, UNITS), lambda k: (0, 0)),
            pl.BlockSpec((1, UNITS), lambda k: (0, 0)),
            pl.BlockSpec((UNITS, 1), lambda k: (0, 0)),
            pl.BlockSpec((1, 1), lambda k: (0, 0)),
        ],
        out_specs=pl.BlockSpec((BLK, 1), lambda k: (k, 0)),
        out_shape=jax.ShapeDtypeStruct((N_NODES, 1), jnp.float32),
        scratch_shapes=[
            pltpu.VMEM((G_DIM, UNITS), jnp.bfloat16),
            pltpu.VMEM((D_FEAT, UNITS), jnp.float32),
            pltpu.VMEM((UNITS, UNITS), jnp.float32),
            pltpu.VMEM((UNITS, UNITS), jnp.float32),
            pltpu.VMEM((1, UNITS), jnp.float32),
            pltpu.VMEM((1, UNITS), jnp.float32),
        ],
        compiler_params=pltpu.CompilerParams(
            dimension_semantics=("arbitrary",)),
    )(inputs, idx3, global_features, W0, row(b0), row(gamma0), row(beta0),
      W1, row(b1), row(gamma1), row(beta1), W2, b2.reshape(1, 1))

    att = _sc_softmax(logits.reshape(N_NODES), graph_indices.astype(jnp.int32))
    return att[:, None]


# speedup vs baseline: 1.0575x; 1.0137x over previous
import functools

import jax
import jax.numpy as jnp
from jax import lax
from jax.experimental import pallas as pl
from jax.experimental.pallas import tpu as pltpu
from jax.experimental.pallas import tpu_sc as plsc

N_NODES = 50000
D_FEAT = 256
G_DIM = 256
NUM_GRAPHS = 256
UNITS = 256
GROUPS = 8
EPS = 1e-3

BLK = 5000
NBLK = N_NODES // BLK
NEG = -1e30


def _pass1_body(x_ref, idx_ref, glob_ref, w0_ref, b0_ref, g0_ref, be0_ref,
                w1_ref, b1_ref, g1_ref, be1_ref, w2_ref, b2_ref,
                logits_ref,
                cg_sc, w0a_sc, w1_sc, p_sc, b0_sc, b1_sc):
    k = pl.program_id(0)

    @pl.when(k == 0)
    def _init():
        gs = UNITS // GROUPS
        ri = jax.lax.broadcasted_iota(jnp.int32, (UNITS, UNITS), 0) // gs
        ci = jax.lax.broadcasted_iota(jnp.int32, (UNITS, UNITS), 1) // gs
        p = (ri == ci).astype(jnp.float32) * (1.0 / gs)
        p_sc[...] = p
        cg = jnp.dot(glob_ref[...], w0_ref[D_FEAT:, :],
                     preferred_element_type=jnp.float32)
        cg_sc[...] = (cg - jnp.dot(cg, p, preferred_element_type=jnp.float32)
                      ).astype(jnp.bfloat16)
        w0a = w0_ref[:D_FEAT, :]
        w0a_sc[...] = w0a - jnp.dot(w0a, p, preferred_element_type=jnp.float32)
        w1 = w1_ref[...]
        w1_sc[...] = w1 - jnp.dot(w1, p, preferred_element_type=jnp.float32)
        b0_sc[...] = b0_ref[...] - jnp.dot(b0_ref[...], p,
                                           preferred_element_type=jnp.float32)
        b1_sc[...] = b1_ref[...] - jnp.dot(b1_ref[...], p,
                                           preferred_element_type=jnp.float32)

    x = x_ref[...]
    idx = idx_ref[0]
    oh_b = idx == jax.lax.broadcasted_iota(jnp.int32, (BLK, NUM_GRAPHS), 1)
    oh = oh_b.astype(jnp.bfloat16)

    hc = (jnp.dot(x, w0a_sc[...], preferred_element_type=jnp.float32)
          + jnp.dot(oh, cg_sc[...], preferred_element_type=jnp.float32)
          + b0_sc[...])
    var = jnp.dot(hc * hc, p_sc[...], preferred_element_type=jnp.float32)
    h = jnp.maximum(hc * jax.lax.rsqrt(var + EPS) * g0_ref[...] + be0_ref[...],
                    0.0)
    hc = jnp.dot(h, w1_sc[...], preferred_element_type=jnp.float32) + b1_sc[...]
    var = jnp.dot(hc * hc, p_sc[...], preferred_element_type=jnp.float32)
    h = jnp.maximum(hc * jax.lax.rsqrt(var + EPS) * g1_ref[...] + be1_ref[...],
                    0.0)
    l = jnp.dot(h, w2_ref[...], preferred_element_type=jnp.float32) + b2_ref[0, 0]
    logits_ref[...] = l


_SC_NW = 32
_SC_NS = 16
_SC_LANES = 16
_SC_PAD = 50176
_SC_CHUNK = _SC_PAD // _SC_NW
_SC_SCHUNK = _SC_PAD // _SC_NS
_SC_UNROLL = 7


def _sc_softmax_body(l_hbm, idx_hbm, att_hbm,
                     l_v, idx_v, att_v, s_v, l2_v, idx2_v, stab_v, red_v,
                     shared_v, sem_l, sem_i, sem_l2, sem_i2):
    cid = lax.axis_index("c")
    sid = lax.axis_index("s")
    wid = sid * 2 + cid
    base = wid * _SC_CHUNK
    sbase = sid * _SC_SCHUNK

    cp_l = pltpu.async_copy(l_hbm.at[pl.ds(base, _SC_CHUNK)], l_v, sem_l)
    cp_i = pltpu.async_copy(idx_hbm.at[pl.ds(base, _SC_CHUNK)], idx_v, sem_i)
    cp_l2 = pltpu.async_copy(l_hbm.at[pl.ds(sbase, _SC_SCHUNK)], l2_v, sem_l2)
    cp_i2 = pltpu.async_copy(idx_hbm.at[pl.ds(sbase, _SC_SCHUNK)], idx2_v,
                             sem_i2)

    for i in range(NUM_GRAPHS // _SC_LANES):
        stab_v[pl.ds(i * _SC_LANES, _SC_LANES)] = jnp.zeros((_SC_LANES,),
                                                            jnp.float32)
    cp_l2.wait()
    cp_i2.wait()

    def sum_step(j, _):
        for u in range(_SC_UNROLL):
            sl = pl.ds((j * _SC_UNROLL + u) * _SC_LANES, _SC_LANES)
            e = jnp.exp(l2_v[sl])
            plsc.addupdate_scatter(stab_v, [idx2_v[sl]], e)
        return 0

    lax.fori_loop(0, _SC_SCHUNK // _SC_LANES // _SC_UNROLL, sum_step, 0)

    pltpu.sync_copy(stab_v, shared_v.at[sid])
    plsc.subcore_barrier()
    pltpu.sync_copy(shared_v, red_v)
    for i in range(NUM_GRAPHS // _SC_LANES):
        sl = pl.ds(i * _SC_LANES, _SC_LANES)
        acc = red_v[0, sl]
        for r in range(1, _SC_NS):
            acc = acc + red_v[r, sl]
        s_v[sl] = acc

    cp_l.wait()
    cp_i.wait()

    def norm_step(j, _):
        for u in range(_SC_UNROLL):
            sl = pl.ds((j * _SC_UNROLL + u) * _SC_LANES, _SC_LANES)
            sg = plsc.load_gather(s_v, [idx_v[sl]])
            att_v[sl] = jnp.exp(l_v[sl]) / sg
        return 0

    lax.fori_loop(0, _SC_CHUNK // _SC_LANES // _SC_UNROLL, norm_step, 0)
    pltpu.sync_copy(att_v, att_hbm.at[pl.ds(base, _SC_CHUNK)])


_sc_softmax = pl.kernel(
    _sc_softmax_body,
    out_type=jax.ShapeDtypeStruct((_SC_PAD,), jnp.float32),
    mesh=plsc.VectorSubcoreMesh(core_axis_name="c", subcore_axis_name="s"),
    scratch_types=[
        pltpu.VMEM((_SC_CHUNK,), jnp.float32),
        pltpu.VMEM((_SC_CHUNK,), jnp.int32),
        pltpu.VMEM((_SC_CHUNK,), jnp.float32),
        pltpu.VMEM((NUM_GRAPHS,), jnp.float32),
        pltpu.VMEM((_SC_SCHUNK,), jnp.float32),
        pltpu.VMEM((_SC_SCHUNK,), jnp.int32),
        pltpu.VMEM((NUM_GRAPHS,), jnp.float32),
        pltpu.VMEM((_SC_NS, NUM_GRAPHS), jnp.float32),
        pltpu.VMEM_SHARED((_SC_NS, NUM_GRAPHS), jnp.float32),
        pltpu.SemaphoreType.DMA,
        pltpu.SemaphoreType.DMA,
        pltpu.SemaphoreType.DMA,
        pltpu.SemaphoreType.DMA,
    ],
    compiler_params=pltpu.CompilerParams(needs_layout_passes=False),
)


@jax.jit
def kernel(inputs, graph_indices, global_features, W0, b0, gamma0, beta0,
           W1, b1, gamma1, beta1, W2, b2):
    idx3 = graph_indices.astype(jnp.int32).reshape(NBLK, BLK, 1)
    row = lambda v: v.reshape(1, -1)

    logits = pl.pallas_call(
        _pass1_body,
        grid=(NBLK,),
        in_specs=[
            pl.BlockSpec((BLK, D_FEAT), lambda k: (k, 0)),
            pl.BlockSpec((1, BLK, 1), lambda k: (k, 0, 0)),
            pl.BlockSpec((NUM_GRAPHS, G_DIM), lambda k: (0, 0)),
            pl.BlockSpec((D_FEAT + G_DIM, UNITS), lambda k: (0, 0)),
            pl.BlockSpec((1, UNITS), lambda k: (0, 0)),
            pl.BlockSpec((1, UNITS), lambda k: (0, 0)),
            pl.BlockSpec((1, UNITS), lambda k: (0, 0)),
            pl.BlockSpec((UNITS, UNITS), lambda k: (0, 0)),
            pl.BlockSpec((1, UNITS), lambda k: (0, 0)),
            pl.BlockSpec((1, UNITS), lambda k: (0, 0)),
            pl.BlockSpec((1, UNITS), lambda k: (0, 0)),
            pl.BlockSpec((UNITS, 1), lambda k: (0, 0)),
            pl.BlockSpec((1, 1), lambda k: (0, 0)),
        ],
        out_specs=pl.BlockSpec((BLK, 1), lambda k: (k, 0)),
        out_shape=jax.ShapeDtypeStruct((N_NODES, 1), jnp.float32),
        scratch_shapes=[
            pltpu.VMEM((G_DIM, UNITS), jnp.bfloat16),
            pltpu.VMEM((D_FEAT, UNITS), jnp.float32),
            pltpu.VMEM((UNITS, UNITS), jnp.float32),
            pltpu.VMEM((UNITS, UNITS), jnp.float32),
            pltpu.VMEM((1, UNITS), jnp.float32),
            pltpu.VMEM((1, UNITS), jnp.float32),
        ],
        compiler_params=pltpu.CompilerParams(
            dimension_semantics=("arbitrary",)),
    )(inputs, idx3, global_features, W0, row(b0), row(gamma0), row(beta0),
      W1, row(b1), row(gamma1), row(beta1), W2, b2.reshape(1, 1))

    l_pad = jnp.pad(logits.reshape(N_NODES), (0, _SC_PAD - N_NODES),
                    constant_values=-1e4)
    i_pad = jnp.pad(graph_indices.astype(jnp.int32), (0, _SC_PAD - N_NODES))
    att = _sc_softmax(l_pad, i_pad)
    return att[:N_NODES, None]


# speedup vs baseline: 1.0584x; 1.0008x over previous
import jax
import jax.numpy as jnp
from jax import lax
from jax.experimental import pallas as pl
from jax.experimental.pallas import tpu as pltpu
from jax.experimental.pallas import tpu_sc as plsc

N_NODES = 50000
D_FEAT = 256
G_DIM = 256
NUM_GRAPHS = 256
UNITS = 256
GROUPS = 8
EPS = 1e-3

BLK = 5000
NBLK = N_NODES // BLK


def _pass1_body(x_ref, idx_ref, glob_ref, w0_ref, b0_ref, g0_ref, be0_ref,
                w1_ref, b1_ref, g1_ref, be1_ref, w2_ref, b2_ref,
                logits_ref,
                cg_sc, w0a_sc, w1_sc, p_sc, b0_sc, b1_sc):
    k = pl.program_id(0)

    @pl.when(k == 0)
    def _init():
        gs = UNITS // GROUPS
        ri = jax.lax.broadcasted_iota(jnp.int32, (UNITS, UNITS), 0) // gs
        ci = jax.lax.broadcasted_iota(jnp.int32, (UNITS, UNITS), 1) // gs
        p = (ri == ci).astype(jnp.float32) * (1.0 / gs)
        p_sc[...] = p
        cg = jnp.dot(glob_ref[...], w0_ref[D_FEAT:, :],
                     preferred_element_type=jnp.float32)
        cg_sc[...] = (cg - jnp.dot(cg, p, preferred_element_type=jnp.float32)
                      ).astype(jnp.bfloat16)
        w0a = w0_ref[:D_FEAT, :]
        w0a_sc[...] = w0a - jnp.dot(w0a, p, preferred_element_type=jnp.float32)
        w1 = w1_ref[...]
        w1_sc[...] = w1 - jnp.dot(w1, p, preferred_element_type=jnp.float32)
        b0_sc[...] = b0_ref[...] - jnp.dot(b0_ref[...], p,
                                           preferred_element_type=jnp.float32)
        b1_sc[...] = b1_ref[...] - jnp.dot(b1_ref[...], p,
                                           preferred_element_type=jnp.float32)

    x = x_ref[...]
    idx = idx_ref[0]
    oh_b = idx == jax.lax.broadcasted_iota(jnp.int32, (BLK, NUM_GRAPHS), 1)
    oh = oh_b.astype(jnp.bfloat16)

    hc = (jnp.dot(x, w0a_sc[...], preferred_element_type=jnp.float32)
          + jnp.dot(oh, cg_sc[...], preferred_element_type=jnp.float32)
          + b0_sc[...])
    var = jnp.dot(hc * hc, p_sc[...], preferred_element_type=jnp.float32)
    h = jnp.maximum(hc * jax.lax.rsqrt(var + EPS) * g0_ref[...] + be0_ref[...],
                    0.0)
    hc = jnp.dot(h, w1_sc[...], preferred_element_type=jnp.float32) + b1_sc[...]
    var = jnp.dot(hc * hc, p_sc[...], preferred_element_type=jnp.float32)
    h = jnp.maximum(hc * jax.lax.rsqrt(var + EPS) * g1_ref[...] + be1_ref[...],
                    0.0)
    l = jnp.dot(h, w2_ref[...], preferred_element_type=jnp.float32) + b2_ref[0, 0]
    logits_ref[...] = l


_SC_NW = 32
_SC_NS = 16
_SC_LANES = 16
_SC_PAD = 50176
_SC_CHUNK = _SC_PAD // _SC_NW
_SC_SCHUNK = _SC_PAD // _SC_NS
_SC_UNROLL = 7


def _sc_softmax_body(l_hbm, idx_hbm, att_hbm,
                     l_v, idx_v, att_v, s_v, l2_v, idx2_v, stab_v, red_v,
                     shared_v, sem_l, sem_i, sem_l2, sem_i2):
    cid = lax.axis_index("c")
    sid = lax.axis_index("s")
    wid = sid * 2 + cid
    base = wid * _SC_CHUNK
    sbase = sid * _SC_SCHUNK

    cp_l = pltpu.async_copy(l_hbm.at[pl.ds(base, _SC_CHUNK)], l_v, sem_l)
    cp_i = pltpu.async_copy(idx_hbm.at[pl.ds(base, _SC_CHUNK)], idx_v, sem_i)
    cp_l2 = pltpu.async_copy(l_hbm.at[pl.ds(sbase, _SC_SCHUNK)], l2_v, sem_l2)
    cp_i2 = pltpu.async_copy(idx_hbm.at[pl.ds(sbase, _SC_SCHUNK)], idx2_v,
                             sem_i2)

    for i in range(NUM_GRAPHS // _SC_LANES):
        stab_v[pl.ds(i * _SC_LANES, _SC_LANES)] = jnp.zeros((_SC_LANES,),
                                                            jnp.float32)
    cp_l2.wait()
    cp_i2.wait()

    def sum_step(j, _):
        for u in range(_SC_UNROLL):
            sl = pl.ds((j * _SC_UNROLL + u) * _SC_LANES, _SC_LANES)
            e = jnp.exp(l2_v[sl])
            plsc.addupdate_scatter(stab_v, [idx2_v[sl]], e)
        return 0

    lax.fori_loop(0, _SC_SCHUNK // _SC_LANES // _SC_UNROLL, sum_step, 0)

    pltpu.sync_copy(stab_v, shared_v.at[sid])
    plsc.subcore_barrier()
    pltpu.sync_copy(shared_v, red_v)
    for i in range(NUM_GRAPHS // _SC_LANES):
        sl = pl.ds(i * _SC_LANES, _SC_LANES)
        acc = red_v[0, sl]
        for r in range(1, _SC_NS):
            acc = acc + red_v[r, sl]
        s_v[sl] = acc

    cp_l.wait()
    cp_i.wait()

    def norm_step(j, _):
        for u in range(_SC_UNROLL):
            sl = pl.ds((j * _SC_UNROLL + u) * _SC_LANES, _SC_LANES)
            sg = plsc.load_gather(s_v, [idx_v[sl]])
            att_v[sl] = jnp.exp(l_v[sl]) / sg
        return 0

    lax.fori_loop(0, _SC_CHUNK // _SC_LANES // _SC_UNROLL, norm_step, 0)
    pltpu.sync_copy(att_v, att_hbm.at[pl.ds(base, _SC_CHUNK)])


_sc_softmax = pl.kernel(
    _sc_softmax_body,
    out_type=jax.ShapeDtypeStruct((_SC_PAD,), jnp.float32),
    mesh=plsc.VectorSubcoreMesh(core_axis_name="c", subcore_axis_name="s"),
    scratch_types=[
        pltpu.VMEM((_SC_CHUNK,), jnp.float32),
        pltpu.VMEM((_SC_CHUNK,), jnp.int32),
        pltpu.VMEM((_SC_CHUNK,), jnp.float32),
        pltpu.VMEM((NUM_GRAPHS,), jnp.float32),
        pltpu.VMEM((_SC_SCHUNK,), jnp.float32),
        pltpu.VMEM((_SC_SCHUNK,), jnp.int32),
        pltpu.VMEM((NUM_GRAPHS,), jnp.float32),
        pltpu.VMEM((_SC_NS, NUM_GRAPHS), jnp.float32),
        pltpu.VMEM_SHARED((_SC_NS, NUM_GRAPHS), jnp.float32),
        pltpu.SemaphoreType.DMA,
        pltpu.SemaphoreType.DMA,
        pltpu.SemaphoreType.DMA,
        pltpu.SemaphoreType.DMA,
    ],
    compiler_params=pltpu.CompilerParams(needs_layout_passes=False),
)


@jax.jit
def kernel(inputs, graph_indices, global_features, W0, b0, gamma0, beta0,
           W1, b1, gamma1, beta1, W2, b2):
    idx3 = graph_indices.astype(jnp.int32).reshape(NBLK, BLK, 1)
    row = lambda v: v.reshape(1, -1)

    logits = pl.pallas_call(
        _pass1_body,
        grid=(NBLK,),
        in_specs=[
            pl.BlockSpec((BLK, D_FEAT), lambda k: (k, 0)),
            pl.BlockSpec((1, BLK, 1), lambda k: (k, 0, 0)),
            pl.BlockSpec((NUM_GRAPHS, G_DIM), lambda k: (0, 0)),
            pl.BlockSpec((D_FEAT + G_DIM, UNITS), lambda k: (0, 0)),
            pl.BlockSpec((1, UNITS), lambda k: (0, 0)),
            pl.BlockSpec((1, UNITS), lambda k: (0, 0)),
            pl.BlockSpec((1, UNITS), lambda k: (0, 0)),
            pl.BlockSpec((UNITS, UNITS), lambda k: (0, 0)),
            pl.BlockSpec((1, UNITS), lambda k: (0, 0)),
            pl.BlockSpec((1, UNITS), lambda k: (0, 0)),
            pl.BlockSpec((1, UNITS), lambda k: (0, 0)),
            pl.BlockSpec((UNITS, 1), lambda k: (0, 0)),
            pl.BlockSpec((1, 1), lambda k: (0, 0)),
        ],
        out_specs=pl.BlockSpec((BLK, 1), lambda k: (k, 0)),
        out_shape=jax.ShapeDtypeStruct((N_NODES, 1), jnp.float32),
        scratch_shapes=[
            pltpu.VMEM((G_DIM, UNITS), jnp.bfloat16),
            pltpu.VMEM((D_FEAT, UNITS), jnp.float32),
            pltpu.VMEM((UNITS, UNITS), jnp.float32),
            pltpu.VMEM((UNITS, UNITS), jnp.float32),
            pltpu.VMEM((1, UNITS), jnp.float32),
            pltpu.VMEM((1, UNITS), jnp.float32),
        ],
        compiler_params=pltpu.CompilerParams(
            dimension_semantics=("arbitrary",)),
    )(inputs, idx3, global_features, W0, row(b0), row(gamma0), row(beta0),
      W1, row(b1), row(gamma1), row(beta1), W2, b2.reshape(1, 1))

    l_pad = jnp.pad(logits.reshape(N_NODES), (0, _SC_PAD - N_NODES),
                    constant_values=-1e4)
    i_pad = jnp.pad(graph_indices.astype(jnp.int32), (0, _SC_PAD - N_NODES))
    att = _sc_softmax(l_pad, i_pad)
    return att[:N_NODES, None]
